# baseline (speedup 1.0000x reference)
import functools
import numpy as _np
import jax
import jax.numpy as jnp
from jax import lax
from jax.experimental import pallas as pl
from jax.experimental.pallas import tpu as pltpu
from jax.experimental.pallas import tpu_sc as plsc

N_IN = 10
N_OUT = 64
M_ROWS = 16384
NVARS = 2 ** N_IN - 2

NCB = 4
NRB = 8
ROWS_PER_TILE = M_ROWS // NRB
GROUPS = ROWS_PER_TILE // 16
CW = N_OUT // NCB


def _batcher_pairs(n):
    pairs = []
    p2 = 1
    while p2 < n:
        p2 *= 2

    def compare(a, b):
        if a < n and b < n:
            pairs.append((a, b))

    def merge(lo, hi, r):
        step = r * 2
        if step < hi - lo:
            merge(lo, hi, step)
            merge(lo + r, hi, step)
            for i in range(lo + r, hi - r, step):
                compare(i, i + r)
        else:
            compare(lo, lo + r)

    def sort(lo, hi):
        if hi - lo >= 1:
            mid = lo + (hi - lo) // 2
            sort(lo, mid)
            sort(mid + 1, hi)
            merge(lo, hi, 1)

    sort(0, p2 - 1)
    return pairs


_PAIRS = _batcher_pairs(N_IN)


def _level_order():
    import numpy as np
    setlist = []
    spans = []
    for k in range(1, N_IN):
        sets = [s for s in range(1, NVARS + 1) if bin(s).count("1") == k]
        start = len(setlist)
        setlist.extend(sets)
        setlist.extend([1023] * ((-len(sets)) % 16))
        spans.append((start // 16, len(setlist) // 16))
    return np.asarray(setlist, dtype=np.int32), spans


_SETLIST, _LEVEL_SPANS = _level_order()
NSETP = len(_SETLIST)


def _round_bf16(x):
    u = plsc.bitcast(x, jnp.uint32)
    lsb = jnp.bitwise_and(
        lax.shift_right_logical(u, jnp.full((16,), 16, jnp.uint32)),
        jnp.full((16,), 1, jnp.uint32))
    u = u + lsb + jnp.full((16,), 0x7FFF, jnp.uint32)
    u = jnp.bitwise_and(u, jnp.full((16,), 0xFFFF0000, jnp.uint32))
    return plsc.bitcast(u, jnp.float32)

_mesh = plsc.VectorSubcoreMesh(core_axis_name="c", subcore_axis_name="s")


@functools.partial(
    pl.kernel,
    out_type=jax.ShapeDtypeStruct((M_ROWS, N_OUT), jnp.float32),
    mesh=_mesh,
    scratch_types=[
        pltpu.VMEM(((NVARS + 2) * CW,), jnp.float32),
        pltpu.VMEM(((NVARS + 2) * CW,), jnp.float32),
        pltpu.VMEM((NSETP * CW,), jnp.float32),
        pltpu.VMEM((NSETP,), jnp.int32),
        pltpu.VMEM((ROWS_PER_TILE * N_IN,), jnp.float32),
        pltpu.VMEM((ROWS_PER_TILE, CW), jnp.float32),
        pltpu.SemaphoreType.DMA,
    ],
    compiler_params=pltpu.CompilerParams(needs_layout_passes=False,
                                         use_tc_tiling_on_sc=False),
)
def _choquet_sc(in_hbm, vars_hbm, slist_hbm, out_hbm, T, Tc, chi, slist,
                xin, ob, sem):
    wid = lax.axis_index("c") * 16 + lax.axis_index("s")
    rb = wid // NCB
    cb = lax.rem(wid, NCB)
    r0 = rb * ROWS_PER_TILE

    in_copy = pltpu.make_async_copy(
        in_hbm.at[pl.ds(r0 * N_IN, ROWS_PER_TILE * N_IN)], xin, sem)
    in_copy.start()
    pltpu.sync_copy(vars_hbm.at[cb], chi)
    pltpu.sync_copy(slist_hbm, slist)

    iota = lax.iota(jnp.int32, 16)
    zeros = jnp.zeros((16,), jnp.float32)
    ones = jnp.ones((16,), jnp.float32)
    fifteen = jnp.full((16,), 15, jnp.int32)
    one_i = jnp.full((16,), 1, jnp.int32)

    T[pl.ds(0, 16)] = zeros

    for blo, bhi in _LEVEL_SPANS:
        @plsc.parallel_loop(blo, bhi, 1)
        def dp_batch(bi):
            s_vec = slist[pl.ds(bi * 16, 16)]
            s16 = lax.shift_left(s_vec, 4)
            addrs = []
            for b in range(N_IN):
                bit = jnp.bitwise_and(lax.shift_right_logical(s_vec, b),
                                      one_i)
                addrs.append(
                    jnp.where(bit == 1,
                              lax.shift_left(s_vec - (1 << b), 4), 0))
            chib = lax.shift_left(iota + bi * 16, 4)
            for c in range(CW):
                colv = jnp.bitwise_and(iota + c, fifteen)
                ls = [plsc.load_gather(T, [addrs[b] + colv])
                      for b in range(N_IN)]
                while len(ls) > 1:
                    ls = [jnp.maximum(ls[i], ls[i + 1])
                          for i in range(0, len(ls) - 1, 2)] + \
                         (ls[-1:] if len(ls) % 2 else [])
                cv = jnp.abs(plsc.load_gather(chi, [chib + colv]))
                row = ls[0] + cv
                plsc.store_scatter(T, [s16 + colv], row)
                plsc.store_scatter(Tc, [s16 + colv],
                                   _round_bf16(jnp.minimum(row, 1.0)))

    Tc[pl.ds((NVARS + 1) * CW, 16)] = ones

    in_copy.wait()

    i10 = iota * N_IN

    @plsc.parallel_loop(0, GROUPS, 1, unroll=2)
    def row_group(g):
        base10 = i10 + g * (16 * N_IN)
        xs = [plsc.load_gather(xin, [base10 + f]) for f in range(N_IN)]
        ps = [jnp.full((16,), 1 << f, jnp.int32) for f in range(N_IN)]
        for a, b in _PAIRS:
            swap = xs[a] < xs[b]
            hi = jnp.maximum(xs[a], xs[b])
            lo = jnp.minimum(xs[a], xs[b])
            pa = jnp.where(swap, ps[b], ps[a])
            pb = jnp.where(swap, ps[a], ps[b])
            xs[a], xs[b] = hi, lo
            ps[a], ps[b] = pa, pb
        cums = [ps[0]]
        for r in range(1, N_IN):
            cums.append(cums[-1] + ps[r])
        a16 = [lax.shift_left(c, 4) for c in cums]
        ds = [_round_bf16(xs[r] - xs[r + 1]) for r in range(N_IN - 1)]
        ds.append(_round_bf16(xs[N_IN - 1]))
        rows16 = iota + g * 16
        for j in range(CW):
            colv = jnp.bitwise_and(iota + j, fifteen)
            acc = ds[N_IN - 1] + ds[0] * plsc.load_gather(Tc, [a16[0] + colv])
            for r in range(1, N_IN - 1):
                acc = acc + ds[r] * plsc.load_gather(Tc, [a16[r] + colv])
            plsc.store_scatter(ob, [rows16, colv], acc)

    pltpu.sync_copy(ob, out_hbm.at[pl.ds(r0, ROWS_PER_TILE),
                                   pl.ds(cb * CW, CW)])


def kernel(inputs, vars):
    in_flat = inputs.reshape(-1)
    perm = jnp.asarray(_np.where(_SETLIST == 1023, 0, _SETLIST - 1))
    vars_lo = vars[perm]
    vars_blk = vars_lo.reshape(NSETP, NCB, CW).transpose(1, 0, 2).reshape(
        NCB, -1)
    return _choquet_sc(in_flat, vars_blk, jnp.asarray(_SETLIST))
